# baseline (device time: 23013 ns/iter reference)
import jax
import jax.numpy as jnp
from jax import lax
from jax.experimental import pallas as pl
from jax.experimental.pallas import tpu as pltpu

CHUNK_ROWS = [32] * 14 + [16] * 4
N_CHUNK = len(CHUNK_ROWS)
CHUNK_OFF = [sum(CHUNK_ROWS[:i]) for i in range(N_CHUNK)]
assert sum(CHUNK_ROWS) == 512


def kernel(x):
    m, n = x.shape
    half = m // 2

    def body(x_ref, out_ref, recv1_buf, recv2_buf, sum_buf,
             send1_sems, recv1_sems, send2_sems, recv2_sems, out_sems):
        my_x = lax.axis_index("x")
        my_y = lax.axis_index("y")
        y_peer = (my_x, 1 - my_y)
        x_peer = (1 - my_x, my_y)

        barrier_sem = pltpu.get_barrier_semaphore()
        for nbr in (y_peer, x_peer):
            pl.semaphore_signal(
                barrier_sem, inc=1,
                device_id=nbr, device_id_type=pl.DeviceIdType.MESH,
            )
        pl.semaphore_wait(barrier_sem, 2)

        half_base = my_x * half
        other_base = (1 - my_x) * half

        p1 = []
        for c in range(N_CHUNK):
            r, o = CHUNK_ROWS[c], CHUNK_OFF[c]
            csl = pl.ds(o, r)
            rdma = pltpu.make_async_remote_copy(
                src_ref=x_ref.at[pl.ds(half_base + o, r), :],
                dst_ref=recv1_buf.at[csl, :],
                send_sem=send1_sems.at[c],
                recv_sem=recv1_sems.at[c],
                device_id=y_peer,
                device_id_type=pl.DeviceIdType.MESH,
            )
            rdma.start()
            p1.append(rdma)

        p2 = []
        for c in range(N_CHUNK):
            r, o = CHUNK_ROWS[c], CHUNK_OFF[c]
            csl = pl.ds(o, r)
            p1[c].wait_recv()
            rdma2 = pltpu.make_async_remote_copy(
                src_ref=recv1_buf.at[csl, :],
                dst_ref=recv2_buf.at[csl, :],
                send_sem=send2_sems.at[c],
                recv_sem=recv2_sems.at[c],
                device_id=x_peer,
                device_id_type=pl.DeviceIdType.MESH,
            )
            rdma2.start()
            p2.append(rdma2)

        loc_out = []
        for c in range(N_CHUNK):
            r, o = CHUNK_ROWS[c], CHUNK_OFF[c]
            csl = pl.ds(o, r)
            sl = pl.ds(half_base + o, r)
            sum_buf[sl, :] = x_ref[sl, :] + recv1_buf[csl, :]
            cp = pltpu.make_async_copy(
                sum_buf.at[sl, :], out_ref.at[sl, :], out_sems.at[c],
            )
            cp.start()
            loc_out.append(cp)

        for c in range(N_CHUNK):
            r, o = CHUNK_ROWS[c], CHUNK_OFF[c]
            csl = pl.ds(o, r)
            osl = pl.ds(other_base + o, r)
            recv2 = pltpu.make_async_remote_copy(
                src_ref=recv2_buf.at[csl, :],
                dst_ref=recv2_buf.at[csl, :],
                send_sem=send2_sems.at[c],
                recv_sem=recv2_sems.at[c],
                device_id=x_peer,
                device_id_type=pl.DeviceIdType.MESH,
            )
            recv2.wait_recv()
            sum_buf[osl, :] = x_ref[osl, :] + recv2_buf[csl, :]
            cp = pltpu.make_async_copy(
                sum_buf.at[osl, :], out_ref.at[osl, :],
                out_sems.at[N_CHUNK + c],
            )
            cp.start()
            loc_out.append(cp)

        for c in range(N_CHUNK):
            p1[c].wait_send()
            p2[c].wait_send()
        for cp in loc_out:
            cp.wait()

    return pl.pallas_call(
        body,
        out_shape=jax.ShapeDtypeStruct((m, n), x.dtype),
        in_specs=[pl.BlockSpec(memory_space=pltpu.VMEM)],
        out_specs=pl.BlockSpec(memory_space=pl.ANY),
        scratch_shapes=[
            pltpu.VMEM((half, n), x.dtype),
            pltpu.VMEM((half, n), x.dtype),
            pltpu.VMEM((m, n), x.dtype),
            pltpu.SemaphoreType.DMA((N_CHUNK,)),
            pltpu.SemaphoreType.DMA((N_CHUNK,)),
            pltpu.SemaphoreType.DMA((N_CHUNK,)),
            pltpu.SemaphoreType.DMA((N_CHUNK,)),
            pltpu.SemaphoreType.DMA((2 * N_CHUNK,)),
        ],
        compiler_params=pltpu.CompilerParams(collective_id=0),
    )(x)


# device time: 19735 ns/iter; 1.1661x vs baseline; 1.1661x over previous
import jax
import jax.numpy as jnp
from jax import lax
from jax.experimental import pallas as pl
from jax.experimental.pallas import tpu as pltpu

N_CHUNK = 16


def kernel(x):
    m, n = x.shape
    half = m // 2
    rows = half // N_CHUNK

    def body(x_ref, out_ref, recv1_buf, recv2_buf,
             send1_sems, recv1_sems, send2_sems, recv2_sems):
        my_x = lax.axis_index("x")
        my_y = lax.axis_index("y")
        y_peer = (my_x, 1 - my_y)
        x_peer = (1 - my_x, my_y)

        barrier_sem = pltpu.get_barrier_semaphore()
        for nbr in (y_peer, x_peer):
            pl.semaphore_signal(
                barrier_sem, inc=1,
                device_id=nbr, device_id_type=pl.DeviceIdType.MESH,
            )
        pl.semaphore_wait(barrier_sem, 2)

        p1, p2 = [], []
        for c in range(N_CHUNK):
            csl = pl.ds(c * rows, rows)
            r1 = pltpu.make_async_remote_copy(
                src_ref=x_ref.at[csl, :],
                dst_ref=recv1_buf.at[csl, :],
                send_sem=send1_sems.at[c],
                recv_sem=recv1_sems.at[c],
                device_id=y_peer,
                device_id_type=pl.DeviceIdType.MESH,
            )
            r1.start()
            p1.append(r1)
            r2 = pltpu.make_async_remote_copy(
                src_ref=x_ref.at[csl, :],
                dst_ref=recv2_buf.at[csl, :],
                send_sem=send2_sems.at[c],
                recv_sem=recv2_sems.at[c],
                device_id=x_peer,
                device_id_type=pl.DeviceIdType.MESH,
            )
            r2.start()
            p2.append(r2)

        for c in range(N_CHUNK):
            p1[c].wait_recv()
            p2[c].wait_recv()
        out_ref[pl.ds(0, half), :] = recv1_buf[...] + recv2_buf[...]
        out_ref[pl.ds(half, half), :] = recv1_buf[...]
        for c in range(N_CHUNK):
            p1[c].wait_send()
            p2[c].wait_send()

    return pl.pallas_call(
        body,
        out_shape=jax.ShapeDtypeStruct((m, n), x.dtype),
        in_specs=[pl.BlockSpec(memory_space=pltpu.VMEM)],
        out_specs=pl.BlockSpec(memory_space=pltpu.VMEM),
        scratch_shapes=[
            pltpu.VMEM((half, n), x.dtype),
            pltpu.VMEM((half, n), x.dtype),
            pltpu.SemaphoreType.DMA((N_CHUNK,)),
            pltpu.SemaphoreType.DMA((N_CHUNK,)),
            pltpu.SemaphoreType.DMA((N_CHUNK,)),
            pltpu.SemaphoreType.DMA((N_CHUNK,)),
        ],
        compiler_params=pltpu.CompilerParams(collective_id=0),
    )(x)
